# baseline (speedup 1.0000x reference)
import functools

import jax
import jax.numpy as jnp
from jax import lax
from jax.experimental import pallas as pl
from jax.experimental.pallas import tpu as pltpu
from jax.experimental.pallas import tpu_sc as plsc

B = 16384
NENT = 1000000
NREL = 100000
ENT_DIM = 64
REL_DIM = 96
H = 32
GAMMA = 12.0
NC, NS, L = 2, 16, 16
NW = NC * NS
BPW = B // NW
CH = 16
NCH = BPW // CH

def _rsqrt(x):
    i = plsc.bitcast(x, jnp.int32)
    i = 0x5F3759DF - (i >> 1)
    y = plsc.bitcast(i, jnp.float32)
    xh = 0.5 * x
    for _ in range(3):
        y = y * (1.5 - xh * y * y)
    return y


def _cst(d):
    return jnp.full((L,), d, jnp.int32)


def _score_body(heads, rels, tails, ent3, remid, out_hbm,
                hidx, ridx, tidx, rdiv0, rdiv1,
                hrows0, rrows0, trows0, hrows1, rrows1, trows1,
                outv, sem0, sem1):
    wid = lax.axis_index("s") * NC + lax.axis_index("c")
    base = wid * BPW
    pltpu.sync_copy(heads.at[pl.ds(base, BPW)], hidx)
    pltpu.sync_copy(rels.at[pl.ds(base, BPW)], ridx)
    pltpu.sync_copy(tails.at[pl.ds(base, BPW)], tidx)

    lane = lax.iota(jnp.int32, L)
    zero = jnp.zeros((L,), jnp.float32)
    divs = [rdiv0, rdiv1]
    rows_bufs = [(hrows0, rrows0, trows0), (hrows1, rrows1, trows1)]
    sems = [sem0, sem1]

    def build_and_fire(c):
        def go(p):
            rd = divs[p]
            hr, rr, tr = rows_bufs[p]
            s = c * CH
            hv = hidx[pl.ds(s, CH)]
            tv = tidx[pl.ds(s, CH)]
            rd[...] = ridx[pl.ds(s, CH)]
            pltpu.async_copy(remid.at[rd], rr, sems[p])
            for j in range(CH):
                pltpu.async_copy(ent3.at[hv[j] >> 3], hr.at[j], sems[p])
                pltpu.async_copy(ent3.at[tv[j] >> 3], tr.at[j], sems[p])
        return go

    def drain(p):
        rd = divs[p]
        hr, rr, tr = rows_bufs[p]
        pltpu.make_async_copy(remid.at[rd], rr, sems[p]).wait()
        pltpu.make_async_copy(ent3.at[pl.ds(0, CH)], hr, sems[p]).wait()
        pltpu.make_async_copy(ent3.at[pl.ds(0, CH)], tr, sems[p]).wait()

    def compute(c, p):
        hr, rr, tr = rows_bufs[p]
        rows = lane
        s = c * CH
        hsub = hidx[pl.ds(s, CH)] & 7
        tsub = tidx[pl.ds(s, CH)] & 7

        def sumsq(ref, sub, lo):
            acc = zero
            for d in range(lo, lo + H):
                x = plsc.load_gather(ref, [rows, sub, _cst(d)])
                acc = acc + x * x
            return acc

        ra = _rsqrt(jnp.maximum(sumsq(hr, hsub, 0), 1e-12))
        rbh = _rsqrt(jnp.maximum(sumsq(hr, hsub, H), 1e-12))
        rat = _rsqrt(jnp.maximum(sumsq(tr, tsub, 0), 1e-12))
        rbt = _rsqrt(jnp.maximum(sumsq(tr, tsub, H), 1e-12))

        acc = zero
        for d in range(H):
            ah = plsc.load_gather(hr, [rows, hsub, _cst(d)])
            bh = plsc.load_gather(hr, [rows, hsub, _cst(H + d)])
            at = plsc.load_gather(tr, [rows, tsub, _cst(d)])
            bt = plsc.load_gather(tr, [rows, tsub, _cst(H + d)])
            m = plsc.load_gather(rr, [rows, _cst(d)])
            s_ = ((ah * ra) * (bt * rbt + 1.0)
                  - (at * rat) * (bh * rbh + 1.0) + m)
            acc = acc + s_ * s_
        norm = acc * _rsqrt(jnp.maximum(acc, 1e-30))
        outv[pl.ds(c * CH, CH)] = GAMMA - norm

    build_and_fire(0)(0)
    build_and_fire(1)(1)

    def pair_body(i, carry):
        c0 = 2 * i
        drain(0)
        compute(c0, 0)

        @pl.when(i < NCH // 2 - 1)
        def _():
            build_and_fire(c0 + 2)(0)
        drain(1)
        compute(c0 + 1, 1)

        @pl.when(i < NCH // 2 - 1)
        def _():
            build_and_fire(c0 + 3)(1)
        return carry

    lax.fori_loop(0, NCH // 2, pair_body, 0)
    pltpu.sync_copy(outv, out_hbm.at[pl.ds(base, BPW)])


@functools.partial(jax.jit, static_argnums=())
def kernel(sample, entity_embedding, relation_embedding):
    sample = sample.astype(jnp.int32)
    heads = sample[:, 0]
    rels = sample[:, 1]
    tails = sample[:, 2]

    mesh = plsc.VectorSubcoreMesh(
        core_axis_name="c", subcore_axis_name="s",
        num_cores=NC, num_subcores=NS)

    def _remid_tc_body(in_ref, o_ref):
        t = in_ref[...].T
        o_ref[...] = pltpu.repeat(t, 4, axis=1)

    remid = pl.pallas_call(
        _remid_tc_body,
        out_shape=jax.ShapeDtypeStruct((NREL, 128), jnp.float32),
        grid=(pl.cdiv(NREL, 512),),
        in_specs=[pl.BlockSpec((H, 512), lambda j: (1, j))],
        out_specs=pl.BlockSpec((512, 128), lambda j: (j, 0)),
    )(relation_embedding.T)

    ent3 = entity_embedding.reshape(NENT // 8, 8, ENT_DIM)
    score = pl.kernel(
        _score_body,
        out_type=jax.ShapeDtypeStruct((B,), jnp.float32),
        mesh=mesh,
        scratch_types=[
            pltpu.VMEM((BPW,), jnp.int32),
            pltpu.VMEM((BPW,), jnp.int32),
            pltpu.VMEM((BPW,), jnp.int32),
            pltpu.VMEM((CH,), jnp.int32),
            pltpu.VMEM((CH,), jnp.int32),
            pltpu.VMEM((CH, 8, ENT_DIM), jnp.float32),
            pltpu.VMEM((CH, 128), jnp.float32),
            pltpu.VMEM((CH, 8, ENT_DIM), jnp.float32),
            pltpu.VMEM((CH, 8, ENT_DIM), jnp.float32),
            pltpu.VMEM((CH, 128), jnp.float32),
            pltpu.VMEM((CH, 8, ENT_DIM), jnp.float32),
            pltpu.VMEM((BPW,), jnp.float32),
            pltpu.SemaphoreType.DMA,
            pltpu.SemaphoreType.DMA,
        ],
        compiler_params=pltpu.CompilerParams(
            needs_layout_passes=False, use_tc_tiling_on_sc=True),
    )(heads, rels, tails, ent3, remid)
    return score.reshape(B, 1)


# speedup vs baseline: 1.0106x; 1.0106x over previous
import functools

import jax
import jax.numpy as jnp
from jax import lax
from jax.experimental import pallas as pl
from jax.experimental.pallas import tpu as pltpu
from jax.experimental.pallas import tpu_sc as plsc

B = 16384
NENT = 1000000
NREL = 100000
ENT_DIM = 64
REL_DIM = 96
H = 32
GAMMA = 12.0
NC, NS, L = 2, 16, 16
NW = NC * NS
BPW = B // NW
CH = 16
NCH = BPW // CH

def _rsqrt(x):
    i = plsc.bitcast(x, jnp.int32)
    i = 0x5F3759DF - (i >> 1)
    y = plsc.bitcast(i, jnp.float32)
    xh = 0.5 * x
    for _ in range(3):
        y = y * (1.5 - xh * y * y)
    return y


def _cst(d):
    return jnp.full((L,), d, jnp.int32)


def _score_body(heads, rels, tails, ent3, remid, out_hbm,
                hidx, ridx, tidx, rdiv0, rdiv1, rdiv2,
                hrows0, rrows0, trows0, hrows1, rrows1, trows1,
                hrows2, rrows2, trows2,
                outv, sem0, sem1, sem2):
    wid = lax.axis_index("s") * NC + lax.axis_index("c")
    base = wid * BPW
    pltpu.sync_copy(heads.at[pl.ds(base, BPW)], hidx)
    pltpu.sync_copy(rels.at[pl.ds(base, BPW)], ridx)
    pltpu.sync_copy(tails.at[pl.ds(base, BPW)], tidx)

    lane = lax.iota(jnp.int32, L)
    zero = jnp.zeros((L,), jnp.float32)
    divs = [rdiv0, rdiv1, rdiv2]
    rows_bufs = [(hrows0, rrows0, trows0), (hrows1, rrows1, trows1),
                 (hrows2, rrows2, trows2)]
    sems = [sem0, sem1, sem2]

    def build_and_fire(c):
        def go(p):
            rd = divs[p]
            hr, rr, tr = rows_bufs[p]
            s = c * CH
            hv = hidx[pl.ds(s, CH)]
            tv = tidx[pl.ds(s, CH)]
            rd[...] = ridx[pl.ds(s, CH)]
            pltpu.async_copy(remid.at[rd], rr, sems[p])
            for j in range(CH):
                pltpu.async_copy(ent3.at[hv[j] >> 3], hr.at[j], sems[p])
                pltpu.async_copy(ent3.at[tv[j] >> 3], tr.at[j], sems[p])
        return go

    def drain(p):
        rd = divs[p]
        hr, rr, tr = rows_bufs[p]
        pltpu.make_async_copy(remid.at[rd], rr, sems[p]).wait()
        pltpu.make_async_copy(ent3.at[pl.ds(0, CH)], hr, sems[p]).wait()
        pltpu.make_async_copy(ent3.at[pl.ds(0, CH)], tr, sems[p]).wait()

    def compute(c, p):
        hr, rr, tr = rows_bufs[p]
        rows = lane
        s = c * CH
        hsub = hidx[pl.ds(s, CH)] & 7
        tsub = tidx[pl.ds(s, CH)] & 7

        def sumsq(ref, sub, lo):
            acc = zero
            for d in range(lo, lo + H):
                x = plsc.load_gather(ref, [rows, sub, _cst(d)])
                acc = acc + x * x
            return acc

        ra = _rsqrt(jnp.maximum(sumsq(hr, hsub, 0), 1e-12))
        rbh = _rsqrt(jnp.maximum(sumsq(hr, hsub, H), 1e-12))
        rat = _rsqrt(jnp.maximum(sumsq(tr, tsub, 0), 1e-12))
        rbt = _rsqrt(jnp.maximum(sumsq(tr, tsub, H), 1e-12))

        acc = zero
        for d in range(H):
            ah = plsc.load_gather(hr, [rows, hsub, _cst(d)])
            bh = plsc.load_gather(hr, [rows, hsub, _cst(H + d)])
            at = plsc.load_gather(tr, [rows, tsub, _cst(d)])
            bt = plsc.load_gather(tr, [rows, tsub, _cst(H + d)])
            m = plsc.load_gather(rr, [rows, _cst(d)])
            s_ = ((ah * ra) * (bt * rbt + 1.0)
                  - (at * rat) * (bh * rbh + 1.0) + m)
            acc = acc + s_ * s_
        norm = acc * _rsqrt(jnp.maximum(acc, 1e-30))
        outv[pl.ds(c * CH, CH)] = GAMMA - norm

    build_and_fire(0)(0)
    build_and_fire(1)(1)
    build_and_fire(2)(2)

    def trip_body(j, carry):
        for k in range(3):
            c = 3 * j + k
            drain(k)
            compute(c, k)

            @pl.when(c + 3 < NCH)
            def _():
                build_and_fire(c + 3)(k)
        return carry

    lax.fori_loop(0, (NCH - 2) // 3, trip_body, 0)
    drain(0)
    compute(NCH - 2, 0)
    drain(1)
    compute(NCH - 1, 1)
    pltpu.sync_copy(outv, out_hbm.at[pl.ds(base, BPW)])


@functools.partial(jax.jit, static_argnums=())
def kernel(sample, entity_embedding, relation_embedding):
    sample = sample.astype(jnp.int32)
    heads = sample[:, 0]
    rels = sample[:, 1]
    tails = sample[:, 2]

    mesh = plsc.VectorSubcoreMesh(
        core_axis_name="c", subcore_axis_name="s",
        num_cores=NC, num_subcores=NS)

    def _remid_tc_body(in_ref, o_ref):
        t = in_ref[...].T
        o_ref[...] = pltpu.repeat(t, 4, axis=1)

    remid = pl.pallas_call(
        _remid_tc_body,
        out_shape=jax.ShapeDtypeStruct((NREL, 128), jnp.float32),
        grid=(pl.cdiv(NREL, 512),),
        in_specs=[pl.BlockSpec((H, 512), lambda j: (1, j))],
        out_specs=pl.BlockSpec((512, 128), lambda j: (j, 0)),
    )(relation_embedding.T)

    ent3 = entity_embedding.reshape(NENT // 8, 8, ENT_DIM)
    score = pl.kernel(
        _score_body,
        out_type=jax.ShapeDtypeStruct((B,), jnp.float32),
        mesh=mesh,
        scratch_types=[
            pltpu.VMEM((BPW,), jnp.int32),
            pltpu.VMEM((BPW,), jnp.int32),
            pltpu.VMEM((BPW,), jnp.int32),
            pltpu.VMEM((CH,), jnp.int32),
            pltpu.VMEM((CH,), jnp.int32),
            pltpu.VMEM((CH,), jnp.int32),
            pltpu.VMEM((CH, 8, ENT_DIM), jnp.float32),
            pltpu.VMEM((CH, 128), jnp.float32),
            pltpu.VMEM((CH, 8, ENT_DIM), jnp.float32),
            pltpu.VMEM((CH, 8, ENT_DIM), jnp.float32),
            pltpu.VMEM((CH, 128), jnp.float32),
            pltpu.VMEM((CH, 8, ENT_DIM), jnp.float32),
            pltpu.VMEM((CH, 8, ENT_DIM), jnp.float32),
            pltpu.VMEM((CH, 128), jnp.float32),
            pltpu.VMEM((CH, 8, ENT_DIM), jnp.float32),
            pltpu.VMEM((BPW,), jnp.float32),
            pltpu.SemaphoreType.DMA,
            pltpu.SemaphoreType.DMA,
            pltpu.SemaphoreType.DMA,
        ],
        compiler_params=pltpu.CompilerParams(
            needs_layout_passes=False, use_tc_tiling_on_sc=True),
    )(heads, rels, tails, ent3, remid)
    return score.reshape(B, 1)
